# baseline (speedup 1.0000x reference)
import functools

import jax
import jax.numpy as jnp
from jax import lax
from jax.experimental import pallas as pl
from jax.experimental.pallas import tpu as pltpu
from jax.experimental.pallas import tpu_sc as plsc

_B, _S, _G = 4, 2048, 4
_VOCAB, _DIM = 100000, 128
_NC, _NS = 2, 16
_NW = _NC * _NS
_ROWS = _B * _S
_RPW = _ROWS // _NW
_BLK = 32
_NBLK = _RPW // _BLK
_NSLOT = 4

_mesh = plsc.VectorSubcoreMesh(core_axis_name="c", subcore_axis_name="s")


@functools.partial(
    pl.kernel,
    mesh=_mesh,
    out_type=jax.ShapeDtypeStruct((_ROWS, _DIM), jnp.float32),
    scratch_types=[pltpu.VMEM((_G * _RPW,), jnp.int32)]
    + [pltpu.VMEM((_BLK, _DIM), jnp.float32) for _ in range(_NSLOT * _G)]
    + [pltpu.VMEM((_BLK, _DIM), jnp.float32) for _ in range(2)]
    + [pltpu.SemaphoreType.DMA] * (1 + _NSLOT + 2),
)
def _embed_sum(x0_hbm, x1_hbm, x2_hbm, x3_hbm, tab_hbm, out_hbm,
               idx_v,
               b00, b01, b02, b03, b10, b11, b12, b13, b20, b21, b22, b23,
               b30, b31, b32, b33,
               ob0, ob1,
               isem, gsem_0, gsem_1, gsem_2, gsem_3, osem_0, osem_1):
    wid = lax.axis_index("s") * _NC + lax.axis_index("c")
    obase = wid * _RPW
    bufs = ((b00, b01, b02, b03), (b10, b11, b12, b13), (b20, b21, b22, b23),
            (b30, b31, b32, b33))
    gsems = (gsem_0, gsem_1, gsem_2, gsem_3)
    obufs = (ob0, ob1)
    osems = (osem_0, osem_1)

    with jax.named_scope("idx_load"):
        iloads = [
            pltpu.async_copy(
                xg.at[pl.ds(wid * _RPW, _RPW)],
                idx_v.at[pl.ds(g * _RPW, _RPW)],
                isem,
            )
            for g, xg in enumerate((x0_hbm, x1_hbm, x2_hbm, x3_hbm))
        ]
        for c in iloads:
            c.wait()

    def start_block(q):
        slot = q % _NSLOT
        return [
            pltpu.async_copy(
                tab_hbm.at[g].at[idx_v.at[pl.ds(g * _RPW + q * _BLK, _BLK)]],
                bufs[slot][g],
                gsems[slot],
            )
            for g in range(_G)
        ]

    pending = [start_block(q) for q in range(_NSLOT)]
    ostores = [None, None]
    for q in range(_NBLK):
        slot = q % _NSLOT
        with jax.named_scope(f"wait{q}"):
            for c in pending.pop(0):
                c.wait()
            if ostores[q % 2] is not None:
                ostores[q % 2].wait()
        b0, b1, b2, b3 = bufs[slot]
        ob = obufs[q % 2]

        with jax.named_scope(f"sum{q}"):
            @plsc.parallel_loop(0, _BLK)
            def _(r, ob=ob, b0=b0, b1=b1, b2=b2, b3=b3):
                for c in range(_DIM // 16):
                    sl = pl.ds(c * 16, 16)
                    ob[r, sl] = (b0[r, sl] + b1[r, sl]) + (b2[r, sl] + b3[r, sl])

        ostores[q % 2] = pltpu.async_copy(
            ob, out_hbm.at[pl.ds(obase + q * _BLK, _BLK)], osems[q % 2]
        )
        if q + _NSLOT < _NBLK:
            pending.append(start_block(q + _NSLOT))

    with jax.named_scope("out_drain"):
        for c in ostores:
            if c is not None:
                c.wait()


def kernel(x, tables):
    xs = [x[:, :, g].reshape(_ROWS) for g in range(_G)]
    out = _embed_sum(*xs, tables)
    return out.reshape(_B, _S, _DIM)
